# Initial kernel scaffold; baseline (speedup 1.0000x reference)
import jax
import jax.numpy as jnp
from jax.experimental import pallas as pl


def kernel(train, label, month_table, day_table, hour_table, type_table):
    raise NotImplementedError("write your pallas kernel here")


# speedup vs baseline: 4.2486x; 4.2486x over previous
import jax
import jax.numpy as jnp
from jax import lax
from jax.experimental import pallas as pl
from jax.experimental.pallas import tpu as pltpu
from jax.experimental.pallas import tpu_sc as plsc

B, L = 4096, 200
N = B * L
NC, NS = 2, 16
NW = NC * NS
PER_W = N // NW
E = 1280
NBLK = PER_W // E
GROUPS = E // 16

MO_OFF, DA_OFF, HO_OFF, TY_OFF = 0, 32, 96, 152
TAB_WORDS = TY_OFF + 101 * 8


def _sc_body(tr_hbm, lb_hbm, mo_hbm, da_hbm, ho_hbm, ty_hbm,
             o1_hbm, o2_hbm, o3_hbm,
             tab_v, tbuf, lbuf, o1b, o2b, o3b):
    c = lax.axis_index("c")
    s = lax.axis_index("s")
    wid = s * NC + c

    pltpu.sync_copy(mo_hbm, tab_v.at[pl.ds(MO_OFF, 13 * 2)])
    pltpu.sync_copy(da_hbm, tab_v.at[pl.ds(DA_OFF, 32 * 2)])
    pltpu.sync_copy(ho_hbm, tab_v.at[pl.ds(HO_OFF, 25 * 2)])
    pltpu.sync_copy(ty_hbm, tab_v.at[pl.ds(TY_OFF, 101 * 8)])

    iota = lax.iota(jnp.int32, 16)
    zero = iota * 0
    five = zero + 5
    two = zero + 2
    base0 = wid * PER_W

    o1_src = [lax.div(iota + 16 * j, five) * 10 + 1 + lax.rem(iota + 16 * j, five)
              for j in range(5)]
    o2_src = [lax.div(iota + 16 * j, two) * 10 + 1 + lax.rem(iota + 16 * j, two)
              for j in range(2)]
    ch = [iota * 10 + cc for cc in (6, 7, 8, 9)]
    p3 = iota * 14

    def group(i, carry):
        t0 = i * 160
        for j in range(5):
            v = plsc.load_gather(tbuf, [o1_src[j] + t0])
            o1b[pl.ds(i * 80 + j * 16, 16)] = v
        for j in range(2):
            v = plsc.load_gather(lbuf, [o2_src[j] + t0])
            o2b[pl.ds(i * 32 + j * 16, 16)] = v
        ti = plsc.load_gather(tbuf, [ch[0] + t0]).astype(jnp.int32)
        mi = plsc.load_gather(tbuf, [ch[1] + t0]).astype(jnp.int32)
        di = plsc.load_gather(tbuf, [ch[2] + t0]).astype(jnp.int32)
        hi = plsc.load_gather(tbuf, [ch[3] + t0]).astype(jnp.int32)
        ma = mi * 2 + MO_OFF
        da = di * 2 + DA_OFF
        ha = hi * 2 + HO_OFF
        ta = ti * 8 + TY_OFF
        bases = [ma, ma, da, da, ha, ha] + [ta] * 8
        o3base = p3 + i * 224
        for j in range(14):
            g = plsc.load_gather(tab_v, [bases[j] + (j & 1 if j < 6 else j - 6)])
            plsc.store_scatter(o3b, [o3base + j], g)
        return carry

    def block(b, carry):
        base = base0 + b * E
        pltpu.sync_copy(tr_hbm.at[pl.ds(base * 10, E * 10)], tbuf)
        pltpu.sync_copy(lb_hbm.at[pl.ds(base * 10, E * 10)], lbuf)
        lax.fori_loop(0, GROUPS, group, 0, unroll=False)
        pltpu.sync_copy(o1b, o1_hbm.at[pl.ds(base * 5, E * 5)])
        pltpu.sync_copy(o2b, o2_hbm.at[pl.ds(base * 2, E * 2)])
        pltpu.sync_copy(o3b, o3_hbm.at[pl.ds(base * 14, E * 14)])
        return carry

    lax.fori_loop(0, NBLK, block, 0, unroll=False)


@jax.jit
def _run(tr, lb, mo, da, ho, ty):
    f = pl.kernel(
        _sc_body,
        out_type=(
            jax.ShapeDtypeStruct((N * 5,), jnp.float32),
            jax.ShapeDtypeStruct((N * 2,), jnp.float32),
            jax.ShapeDtypeStruct((N * 14,), jnp.float32),
        ),
        mesh=plsc.VectorSubcoreMesh(
            core_axis_name="c", subcore_axis_name="s",
            num_cores=NC, num_subcores=NS,
        ),
        compiler_params=pltpu.CompilerParams(needs_layout_passes=False),
        scratch_types=[
            pltpu.VMEM((TAB_WORDS,), jnp.float32),
            pltpu.VMEM((E * 10,), jnp.float32),
            pltpu.VMEM((E * 10,), jnp.float32),
            pltpu.VMEM((E * 5,), jnp.float32),
            pltpu.VMEM((E * 2,), jnp.float32),
            pltpu.VMEM((E * 14,), jnp.float32),
        ],
    )
    return f(tr, lb, mo, da, ho, ty)


def kernel(train, label, month_table, day_table, hour_table, type_table):
    o1, o2, o3 = _run(
        train.reshape(-1), label.reshape(-1),
        month_table.reshape(-1), day_table.reshape(-1),
        hour_table.reshape(-1), type_table.reshape(-1),
    )
    return (o1.reshape(B, L, 5), o2.reshape(B, L, 2), o3.reshape(B, L, 14))


# speedup vs baseline: 51.2025x; 12.0516x over previous
import jax
import jax.numpy as jnp
from jax import lax
from jax.experimental import pallas as pl
from jax.experimental.pallas import tpu as pltpu
from jax.experimental.pallas import tpu_sc as plsc

B, L = 4096, 200
NC, NS = 2, 16
NW = NC * NS
BW = B // NW
NL = 8
NBLK = L // NL
KCH = 8

PLANE_OFF = [0, 13, 26, 58, 90, 115] + [140 + 101 * j for j in range(8)]
CAT_WORDS = 140 + 101 * 8


def _sc_body(tr_hbm, lb_hbm, cat_hbm,
             o1_hbm, o2_hbm, o3_hbm,
             cat_v, ibuf, t15, lb2, o2b, o3b):
    c = lax.axis_index("c")
    s = lax.axis_index("s")
    wid = s * NC + c
    b0 = wid * BW

    pltpu.sync_copy(cat_hbm, cat_v)

    def group(l, k):
        sl = pl.ds(k * 16, 16)
        ti = ibuf[0, l, sl].astype(jnp.int32)
        mi = ibuf[1, l, sl].astype(jnp.int32)
        di = ibuf[2, l, sl].astype(jnp.int32)
        hi = ibuf[3, l, sl].astype(jnp.int32)
        idx = [mi, mi, di, di, hi, hi] + [ti] * 8
        for j in range(14):
            o3b[j, l, sl] = plsc.load_gather(cat_v, [idx[j] + PLANE_OFF[j]])
        o2b[l, 0, sl] = lb2[0, l, sl]
        o2b[l, 1, sl] = lb2[1, l, sl]

    def row(l, carry):
        for k in range(KCH):
            group(l, k)
        return carry

    def block(blk, carry):
        l0 = blk * NL
        cols = pl.ds(b0, BW)
        rows = pl.ds(l0, NL)
        pltpu.sync_copy(tr_hbm.at[pl.ds(6, 4), rows, cols], ibuf)
        pltpu.sync_copy(tr_hbm.at[pl.ds(1, 5), rows, cols], t15)
        pltpu.sync_copy(lb_hbm.at[pl.ds(1, 2), rows, cols], lb2)
        lax.fori_loop(0, NL, row, 0, unroll=False)
        pltpu.sync_copy(t15, o1_hbm.at[pl.ds(0, 5), rows, cols])
        pltpu.sync_copy(o2b, o2_hbm.at[rows, pl.ds(0, 2), cols])
        pltpu.sync_copy(o3b, o3_hbm.at[pl.ds(0, 14), rows, cols])
        return carry

    lax.fori_loop(0, NBLK, block, 0, unroll=False)


@jax.jit
def _run(tr_t, lb_t, cat):
    f = pl.kernel(
        _sc_body,
        out_type=(
            jax.ShapeDtypeStruct((5, L, B), jnp.float32),
            jax.ShapeDtypeStruct((L, 2, B), jnp.float32),
            jax.ShapeDtypeStruct((14, L, B), jnp.float32),
        ),
        mesh=plsc.VectorSubcoreMesh(
            core_axis_name="c", subcore_axis_name="s",
            num_cores=NC, num_subcores=NS,
        ),
        compiler_params=pltpu.CompilerParams(
            needs_layout_passes=False,
            use_tc_tiling_on_sc=True,
        ),
        scratch_types=[
            pltpu.VMEM((CAT_WORDS,), jnp.float32),
            pltpu.VMEM((4, NL, BW), jnp.float32),
            pltpu.VMEM((5, NL, BW), jnp.float32),
            pltpu.VMEM((2, NL, BW), jnp.float32),
            pltpu.VMEM((NL, 2, BW), jnp.float32),
            pltpu.VMEM((14, NL, BW), jnp.float32),
        ],
    )
    return f(tr_t, lb_t, cat)


def kernel(train, label, month_table, day_table, hour_table, type_table):
    tr_t = jnp.transpose(train, (2, 1, 0))
    lb_t = jnp.transpose(label, (2, 1, 0))
    cat = jnp.concatenate([
        month_table.T.reshape(-1), day_table.T.reshape(-1),
        hour_table.T.reshape(-1), type_table.T.reshape(-1),
    ])
    o1_t, o2_t, o3_t = _run(tr_t, lb_t, cat)
    return (
        jnp.transpose(o1_t, (2, 1, 0)),
        jnp.transpose(o2_t, (2, 0, 1)),
        jnp.transpose(o3_t, (2, 1, 0)),
    )
